# baseline (speedup 1.0000x reference)
import functools

import jax
import jax.numpy as jnp
from jax import lax
from jax.experimental import pallas as pl
from jax.experimental.pallas import tpu as pltpu
from jax.experimental.pallas import tpu_sc as plsc

DIM = 32
BLK = 128


@functools.lru_cache(maxsize=None)
def _build(n_flat, n_pre, n_new, vocab):
    info = plsc.get_sparse_core_info()
    nc, ns, lanes = info.num_cores, info.num_subcores, info.num_lanes
    nw = nc * ns
    assert n_flat % (nw * BLK) == 0
    blocks_per_w = n_flat // (nw * BLK)
    dump = n_flat

    mesh = plsc.VectorSubcoreMesh(core_axis_name="c", subcore_axis_name="s")

    @functools.partial(
        pl.kernel,
        out_type=jax.ShapeDtypeStruct((n_flat + 8, DIM), jnp.float32),
        mesh=mesh,
        compiler_params=pltpu.CompilerParams(use_tc_tiling_on_sc=False),
        scratch_types=[
            pltpu.VMEM((blocks_per_w, BLK), jnp.int32),
            pltpu.VMEM((blocks_per_w, BLK), jnp.int32),
            pltpu.VMEM((blocks_per_w, BLK), jnp.int32),
            pltpu.VMEM((blocks_per_w, BLK), jnp.int32),
            pltpu.VMEM((blocks_per_w, BLK), jnp.int32),
            pltpu.VMEM((blocks_per_w, BLK), jnp.int32),
            pltpu.VMEM((BLK, DIM), jnp.float32),
            pltpu.VMEM((BLK, DIM), jnp.float32),
            pltpu.SemaphoreType.DMA,
        ],
    )
    def emb(x_hbm, re_hbm, pre_hbm, new_hbm, out_hbm,
            xv, rv, pidx, nidx, pos_a, pos_b, prebuf, newbuf, sem):
        c = lax.axis_index("c")
        s = lax.axis_index("s")
        wid = s * nc + c
        rbase = wid * blocks_per_w

        pltpu.sync_copy(x_hbm.at[wid], xv)

        def fire(j, _):
            pltpu.async_copy(re_hbm.at[xv.at[j]], rv.at[j], sem)
            return 0

        lax.fori_loop(0, blocks_per_w, fire, 0)

        def drain(j, _):
            pltpu.make_async_copy(re_hbm.at[xv.at[j]], rv.at[j], sem).wait()
            return 0

        lax.fori_loop(0, blocks_per_w, drain, 0)

        lane = lax.iota(jnp.int32, lanes)

        def route(j, _):
            for i in range(BLK // lanes):
                sl = pl.ds(i * lanes, lanes)
                r = rv[j, sl]
                m = r < n_pre
                pidx[j, sl] = jnp.where(m, r, 0)
                nidx[j, sl] = jnp.where(m, 0, r - n_pre)
                g = (rbase + j) * BLK + i * lanes + lane
                pos_a[j, sl] = jnp.where(m, g, dump)
                pos_b[j, sl] = jnp.where(m, dump, g)
            return 0

        lax.fori_loop(0, blocks_per_w, route, 0)

        def move(j, _):
            gp = pltpu.async_copy(pre_hbm.at[pidx.at[j]], prebuf, sem)
            gn = pltpu.async_copy(new_hbm.at[nidx.at[j]], newbuf, sem)
            gp.wait()
            gn.wait()
            sa = pltpu.async_copy(prebuf, out_hbm.at[pos_a.at[j]], sem)
            sb = pltpu.async_copy(newbuf, out_hbm.at[pos_b.at[j]], sem)
            sa.wait()
            sb.wait()
            return 0

        lax.fori_loop(0, blocks_per_w, move, 0)

    return emb


def kernel(x, reordering, pretrained_weight, new_weight):
    b, l = x.shape
    n_flat = b * l
    n_pre = pretrained_weight.shape[0]
    n_new = new_weight.shape[0]
    vocab = reordering.shape[0]
    emb = _build(n_flat, n_pre, n_new, vocab)
    info = plsc.get_sparse_core_info()
    nw = info.num_cores * info.num_subcores
    xf = x.reshape(nw, n_flat // (nw * BLK), BLK)
    out = emb(xf, reordering, pretrained_weight, new_weight)
    return out[:n_flat].reshape(b, l, DIM)
